# baseline (speedup 1.0000x reference)
import jax
import jax.numpy as jnp
from jax.experimental import pallas as pl
from jax.experimental.pallas import tpu as pltpu

_BI = 2000
_BK = 1000
_BL = 2000
_ONCE = pl.Buffered(buffer_count=1)


def _linear_kernel(x_ref, wt_ref, o_ref):
    xb = x_ref[...].astype(jnp.bfloat16)
    o_ref[...] = jnp.dot(
        xb, wt_ref[...], preferred_element_type=jnp.float32
    ).astype(jnp.bfloat16)


def _gcn_mid_kernel(adj_ref, s_ref, b_ref, a_ref, wt_ref, o_ref, acc_ref):
    k = pl.program_id(1)
    part = jnp.dot(
        adj_ref[...].reshape(_BI, _BK),
        s_ref[pl.ds(k * _BK, _BK), :],
        preferred_element_type=jnp.float32,
    )

    @pl.when(k == 0)
    def _():
        acc_ref[...] = part

    @pl.when(k > 0)
    def _():
        acc_ref[...] += part

    @pl.when(k == pl.num_programs(1) - 1)
    def _():
        acc = acc_ref[...] + b_ref[...]
        h = jnp.where(acc >= 0, acc, a_ref[0, 0] * acc).astype(jnp.bfloat16)
        o_ref[...] = jnp.dot(
            h, wt_ref[...], preferred_element_type=jnp.float32
        ).astype(jnp.bfloat16)


def _gcn_out_kernel(adj_ref, s_ref, b_ref, a_ref, o_ref, acc_ref):
    k = pl.program_id(1)
    part = jnp.dot(
        adj_ref[...].reshape(_BI, _BK),
        s_ref[pl.ds(k * _BK, _BK), :],
        preferred_element_type=jnp.float32,
    )

    @pl.when(k == 0)
    def _():
        acc_ref[...] = part

    @pl.when(k > 0)
    def _():
        acc_ref[...] += part

    @pl.when(k == pl.num_programs(1) - 1)
    def _():
        acc = acc_ref[...] + b_ref[...]
        o_ref[...] = jnp.where(acc >= 0, acc, a_ref[0, 0] * acc)


def kernel(x, adj, sparse, W1, b1, a1, W2, b2, a2):
    n, d = x.shape[1], x.shape[2]
    nk = n // _BK
    x2 = x.reshape(n, d)
    adj4 = adj.reshape(n, nk, 1, _BK)
    w1t = W1.T.astype(jnp.bfloat16)
    w2t = W2.T.astype(jnp.bfloat16)
    b1r = b1.astype(jnp.float32).reshape(1, d)
    b2r = b2.astype(jnp.float32).reshape(1, d)
    a1r = jnp.asarray(a1, jnp.float32).reshape(1, 1)
    a2r = jnp.asarray(a2, jnp.float32).reshape(1, 1)

    const = lambda *_: (0, 0)
    row = lambda i: (i, 0)

    s1 = pl.pallas_call(
        _linear_kernel,
        grid=(n // _BL,),
        in_specs=[
            pl.BlockSpec((_BL, d), row),
            pl.BlockSpec((d, d), const),
        ],
        out_specs=pl.BlockSpec((_BL, d), row),
        out_shape=jax.ShapeDtypeStruct((n, d), jnp.bfloat16),
    )(x2, w1t)

    tile = lambda i, k: (i, k, 0, 0)
    const2 = lambda i, k: (0, 0)
    row2 = lambda i, k: (i, 0)
    grid2 = (n // _BI, nk)

    s2 = pl.pallas_call(
        _gcn_mid_kernel,
        grid=grid2,
        in_specs=[
            pl.BlockSpec((_BI, 1, 1, _BK), tile),
            pl.BlockSpec((n, d), const2, pipeline_mode=_ONCE),
            pl.BlockSpec((1, d), const2, pipeline_mode=_ONCE),
            pl.BlockSpec((1, 1), const2, pipeline_mode=_ONCE),
            pl.BlockSpec((d, d), const2, pipeline_mode=_ONCE),
        ],
        out_specs=pl.BlockSpec((_BI, d), row2),
        out_shape=jax.ShapeDtypeStruct((n, d), jnp.bfloat16),
        scratch_shapes=[pltpu.VMEM((_BI, d), jnp.float32)],
        compiler_params=pltpu.CompilerParams(
            vmem_limit_bytes=63 * 1024 * 1024),
    )(adj4, s1, b1r, a1r, w2t)

    h = pl.pallas_call(
        _gcn_out_kernel,
        grid=grid2,
        in_specs=[
            pl.BlockSpec((_BI, 1, 1, _BK), tile),
            pl.BlockSpec((n, d), const2, pipeline_mode=_ONCE),
            pl.BlockSpec((1, d), const2, pipeline_mode=_ONCE),
            pl.BlockSpec((1, 1), const2, pipeline_mode=_ONCE),
        ],
        out_specs=pl.BlockSpec((_BI, d), row2),
        out_shape=jax.ShapeDtypeStruct((n, d), jnp.float32),
        scratch_shapes=[pltpu.VMEM((_BI, d), jnp.float32)],
        compiler_params=pltpu.CompilerParams(
            vmem_limit_bytes=63 * 1024 * 1024),
    )(adj4, s2, b2r, a2r)

    return (h.reshape(1, n, d), h)


# speedup vs baseline: 10.5658x; 10.5658x over previous
import jax
import jax.numpy as jnp
from jax.experimental import pallas as pl
from jax.experimental.pallas import tpu as pltpu

_BI = 400
_BL = 2000


def _linear_kernel(x_ref, wt_ref, o_ref):
    xb = x_ref[...].astype(jnp.bfloat16)
    o_ref[...] = jnp.dot(
        xb, wt_ref[...], preferred_element_type=jnp.float32
    ).astype(jnp.bfloat16)


def _gcn_mid_kernel(adj_ref, s_ref, b_ref, a_ref, wt_ref, o_ref):
    acc = jnp.dot(
        adj_ref[...], s_ref[...],
        preferred_element_type=jnp.float32,
    )
    acc = acc + b_ref[...]
    h = jnp.where(acc >= 0, acc, a_ref[0, 0] * acc).astype(jnp.bfloat16)
    o_ref[...] = jnp.dot(
        h, wt_ref[...], preferred_element_type=jnp.float32
    ).astype(jnp.bfloat16)


def _gcn_out_kernel(adj_ref, s_ref, b_ref, a_ref, o_ref, o2_ref):
    acc = jnp.dot(
        adj_ref[...], s_ref[...],
        preferred_element_type=jnp.float32,
    )
    acc = acc + b_ref[...]
    h = jnp.where(acc >= 0, acc, a_ref[0, 0] * acc)
    o_ref[...] = h
    o2_ref[...] = h


def kernel(x, adj, sparse, W1, b1, a1, W2, b2, a2):
    n, d = x.shape[1], x.shape[2]
    x2 = x.reshape(n, d)
    adj2 = adj.reshape(n, n)
    w1t = W1.T.astype(jnp.bfloat16)
    w2t = W2.T.astype(jnp.bfloat16)
    b1r = b1.astype(jnp.float32).reshape(1, d)
    b2r = b2.astype(jnp.float32).reshape(1, d)
    a1r = jnp.asarray(a1, jnp.float32).reshape(1, 1)
    a2r = jnp.asarray(a2, jnp.float32).reshape(1, 1)

    const = lambda *_: (0, 0)
    row = lambda i: (i, 0)

    s1 = pl.pallas_call(
        _linear_kernel,
        grid=(n // _BL,),
        in_specs=[
            pl.BlockSpec((_BL, d), row),
            pl.BlockSpec((d, d), const),
        ],
        out_specs=pl.BlockSpec((_BL, d), row),
        out_shape=jax.ShapeDtypeStruct((n, d), jnp.bfloat16),
    )(x2, w1t)

    s2 = pl.pallas_call(
        _gcn_mid_kernel,
        grid=(n // _BI,),
        in_specs=[
            pl.BlockSpec((_BI, n), row),
            pl.BlockSpec((n, d), const),
            pl.BlockSpec((1, d), const),
            pl.BlockSpec((1, 1), const),
            pl.BlockSpec((d, d), const),
        ],
        out_specs=pl.BlockSpec((_BI, d), row),
        out_shape=jax.ShapeDtypeStruct((n, d), jnp.bfloat16),
    )(adj2, s1, b1r, a1r, w2t)

    h = pl.pallas_call(
        _gcn_out_kernel,
        grid=(n // _BI,),
        in_specs=[
            pl.BlockSpec((_BI, n), row),
            pl.BlockSpec((n, d), const),
            pl.BlockSpec((1, d), const),
            pl.BlockSpec((1, 1), const),
        ],
        out_specs=[
            pl.BlockSpec((_BI, d), row),
            pl.BlockSpec((_BI, d), row),
        ],
        out_shape=[
            jax.ShapeDtypeStruct((n, d), jnp.float32),
            jax.ShapeDtypeStruct((n, d), jnp.float32),
        ],
    )(adj2, s2, b2r, a2r)

    return (h[0].reshape(1, n, d), h[1])
